# baseline (speedup 1.0000x reference)
import functools

import jax
import jax.numpy as jnp
from jax import lax
from jax.experimental import pallas as pl
from jax.experimental.pallas import tpu as pltpu
from jax.experimental.pallas import tpu_sc as plsc

_V = 5000
_R = 512
_D = 256
_SEQ = 256
_B = 4
_C = 3
_NCLS = (6, 8, 10)
_H = 4
_DH = 64
_E = 131072
_NPAD = 16


def _fused_body(bm, cm, wb, wc, bb2, bc2, vis, wq, wk, wv, wo,
                mask_r, cnt_r, *rest):
    raw = rest[:36]
    out_ref = rest[36]
    (lncnt_s, rule_s, emb_s, w0_s, w1_s, w2p_s, as0_s, ad0_s, as1_s, ad1_s,
     as2_s, ad2_s, b0_s, b1_s, b2_s) = rest[37:]
    f32 = jnp.float32
    i = pl.program_id(0)
    bi = pl.program_id(1)

    @pl.when(jnp.logical_and(i == 0, bi == 0))
    def _():
        w2p_s[...] = jnp.zeros(w2p_s.shape, f32)
        for ref in (as0_s, ad0_s, as1_s, ad1_s, as2_s, ad2_s):
            ref[...] = jnp.zeros(ref.shape, f32)
        for ref in (b0_s, b1_s, b2_s):
            ref[...] = jnp.zeros(ref.shape, f32)

    for ci in range(_C):
        @pl.when(jnp.logical_and(i == ci, bi == 0))
        def _(ci=ci):
            ncls = _NCLS[ci]
            (w0r, w1r, w2r, as0r, ad0r, as1r, ad1r, as2r, ad2r,
             b0r, b1r, b2r) = raw[ci * 12:(ci + 1) * 12]
            w0_s[...] = w0r[...]
            w1_s[...] = w1r[...]
            for hh in range(_H):
                w2p_s[:, hh * _NPAD:hh * _NPAD + ncls] = (
                    w2r[:, hh * ncls:(hh + 1) * ncls])
                as0_s[hh:hh + 1, hh * 64:(hh + 1) * 64] = as0r[hh:hh + 1, :]
                ad0_s[hh:hh + 1, hh * 64:(hh + 1) * 64] = ad0r[hh:hh + 1, :]
                as1_s[hh:hh + 1, hh * 32:(hh + 1) * 32] = as1r[hh:hh + 1, :]
                ad1_s[hh:hh + 1, hh * 32:(hh + 1) * 32] = ad1r[hh:hh + 1, :]
                as2_s[hh:hh + 1, hh * _NPAD:hh * _NPAD + ncls] = (
                    as2r[hh:hh + 1, :])
                ad2_s[hh:hh + 1, hh * _NPAD:hh * _NPAD + ncls] = (
                    ad2r[hh:hh + 1, :])
            b0_s[:, :] = b0r[...]
            b1_s[:, :] = b1r[...]
            b2_s[:, :ncls] = b2r[...]

    @pl.when(jnp.logical_and(i == 0, bi == 0))
    def _():
        acc = jnp.dot(bm[...], wb[...], preferred_element_type=f32)
        acc = acc + jnp.dot(cm[...], wc[...], preferred_element_type=f32)
        rule_s[...] = acc + bb2[...] + bc2[...]

    @pl.when(i == 0)
    def _():
        q = jnp.dot(rule_s[...], wq[...], preferred_element_type=f32)
        visb = vis[0]
        k = jnp.dot(visb, wk[...], preferred_element_type=f32)
        v = jnp.dot(visb, wv[...], preferred_element_type=f32)
        outs = []
        for h in range(_H):
            sl = slice(h * _DH, (h + 1) * _DH)
            logits = lax.dot_general(q[:, sl], k[:, sl],
                                     (((1,), (1,)), ((), ())),
                                     preferred_element_type=f32) * (1.0 / 8.0)
            m = jnp.max(logits, axis=1, keepdims=True)
            p = jnp.exp(logits - m)
            s = jnp.sum(p, axis=1, keepdims=True)
            outs.append(jnp.dot(p / s, v[:, sl], preferred_element_type=f32))
        o = jnp.concatenate(outs, axis=1)
        emb_s[pl.ds(bi, 1)] = jnp.dot(o, wo[...],
                                      preferred_element_type=f32)[None]

    @pl.when(jnp.logical_and(i == 0, bi == 0))
    def _():
        cnt = cnt_r[0]
        for pslab in range(1, cnt_r.shape[0]):
            cnt = cnt + cnt_r[pslab]
        eye = (lax.broadcasted_iota(jnp.int32, (_R, _R), 0) ==
               lax.broadcasted_iota(jnp.int32, (_R, _R), 1)).astype(f32)
        cnt = cnt + eye
        lncnt_s[...] = jnp.where(cnt > 0.0, jnp.log(cnt), -1e30)

    lncnt = lncnt_s[...]

    h = emb_s[pl.ds(bi, 1)][0] * mask_r[0]

    Ws = (w0_s[...], w1_s[...], w2p_s[...])
    As = (as0_s[...], as1_s[...], as2_s[...])
    Ad = (ad0_s[...], ad1_s[...], ad2_s[...])
    Bs = (b0_s[...], b1_s[...], b2_s[...])
    dpads = (64, 32, _NPAD)
    for l in range(3):
        W = Ws[l]
        a_s = As[l]
        a_d = Ad[l]
        bvec = Bs[l]
        dpad = dpads[l]
        xp = jnp.dot(h, W, preferred_element_type=f32)
        ed_all = lax.dot_general(xp, a_d, (((1,), (1,)), ((), ())),
                                 preferred_element_type=f32)
        es_all = lax.dot_general(a_s, xp, (((1,), (1,)), ((), ())),
                                 preferred_element_type=f32)
        ones_col = jnp.ones((_R, 1), f32)
        acc = jnp.zeros((_R, dpad), f32)
        for hh in range(_H):
            xpa = jnp.concatenate(
                [xp[:, hh * dpad:(hh + 1) * dpad], ones_col], axis=1)
            t = ed_all[:, hh:hh + 1] + es_all[hh:hh + 1, :]
            e2 = jnp.maximum(t, 0.2 * t) + lncnt
            wgt = jnp.exp(e2)
            od = jnp.dot(wgt, xpa, preferred_element_type=f32)
            acc = acc + od[:, :dpad] / (od[:, dpad:dpad + 1] + 1e-16)
        h = acc * (1.0 / _H) + bvec
        if l < 2:
            h = jnp.maximum(h, 0.0)

    colsum = jnp.sum(h, axis=0, keepdims=True)

    @pl.when(bi == 0)
    def _():
        out_ref[0] = jnp.zeros((8, _NPAD), f32)

    out_ref[0, pl.ds(bi, 1), :] = colsum

    @pl.when(bi == _B - 1)
    def _():
        o = out_ref[0]
        ncls_i = jnp.where(i == 0, _NCLS[0],
                           jnp.where(i == 1, _NCLS[1], _NCLS[2]))
        col = lax.broadcasted_iota(jnp.int32, (1, _NPAD), 1)
        om = jnp.where(col < ncls_i, o, -1e30)
        m = jnp.max(om, axis=1, keepdims=True)
        lse = jnp.log(jnp.sum(jnp.exp(om - m), axis=1, keepdims=True)) + m
        out_ref[0] = o - lse


_NC = 2
_NS = 16
_EPW = _E // (_NC * _NS)
_NJ = _EPW // 128


def _cnt_sc_body(src_hbm, dst_hbm, zeros_hbm, out_hbm,
                 src_v, dst_v, idx_v, vals_v, cnt_sh, sem):
    c = lax.axis_index("c")
    s = lax.axis_index("s")
    wid = c * _NS + s
    base = wid * _EPW
    sl16k = pl.ds(s * (_R * _R // _NS), _R * _R // _NS)

    pltpu.sync_copy(zeros_hbm.at[sl16k], cnt_sh.at[sl16k])

    pltpu.sync_copy(src_hbm.at[pl.ds(base, _EPW)], src_v)
    pltpu.sync_copy(dst_hbm.at[pl.ds(base, _EPW)], dst_v)

    for j in range(_NJ):
        for q in range(8):
            o = j * 128 + q * 16
            d = dst_v[pl.ds(o, 16)]
            sr = src_v[pl.ds(o, 16)]
            idx_v[j, pl.ds(q * 16, 16)] = d * _R + sr
    for q in range(8):
        vals_v[pl.ds(q * 16, 16)] = jnp.full((16,), 1.0, jnp.float32)

    plsc.subcore_barrier()

    descs = [
        pltpu.async_copy(vals_v, cnt_sh.at[idx_v.at[j]], sem, add=True)
        for j in range(_NJ)
    ]
    for d_ in descs:
        d_.wait()

    plsc.subcore_barrier()

    pltpu.sync_copy(cnt_sh.at[sl16k], out_hbm.at[c, sl16k])


def _build_cnt_sc(edge_index):
    src_hbm = edge_index[0]
    dst_hbm = edge_index[1]
    zeros = jnp.zeros((_R * _R,), jnp.float32)
    mesh = plsc.VectorSubcoreMesh(core_axis_name="c", subcore_axis_name="s",
                                  num_cores=_NC, num_subcores=_NS)
    k = functools.partial(
        pl.kernel,
        out_type=jax.ShapeDtypeStruct((_NC, _R * _R), jnp.float32),
        mesh=mesh,
        scratch_types=[
            pltpu.VMEM((_EPW,), jnp.int32),
            pltpu.VMEM((_EPW,), jnp.int32),
            pltpu.VMEM((_NJ, 128), jnp.int32),
            pltpu.VMEM((128,), jnp.float32),
            pltpu.VMEM_SHARED((_R * _R,), jnp.float32),
            pltpu.SemaphoreType.DMA,
        ],
    )(_cnt_sc_body)
    cnt2 = k(src_hbm, dst_hbm, zeros)
    return cnt2.reshape(_NC, _R, _R)


def kernel(vis_emb, params, basic_multihot, crucial_multihot, mask, edge_index):
    p = params
    f32 = jnp.float32
    edge_index = edge_index.astype(jnp.int32)

    bb2 = p['bb'][None, :]
    bc2 = p['bc'][None, :]
    vis3 = vis_emb.reshape(_B, _SEQ, _D)
    cntp = _build_cnt_sc(edge_index)

    maskT = mask.reshape(_C, _R, 1)

    raws = []
    raw_specs = []
    full = lambda shape: pl.BlockSpec(shape, lambda i, bi: (0,) * len(shape))
    byi = lambda shape: pl.BlockSpec((1,) + shape[1:],
                                     lambda i, bi: (i,) + (0,) * (len(shape) - 1))
    for ci in range(_C):
        ncls = _NCLS[ci]
        raws += [p['g%d_0_W' % ci], p['g%d_1_W' % ci],
                 p['g%d_2_W' % ci],
                 p['g%d_0_as' % ci], p['g%d_0_ad' % ci],
                 p['g%d_1_as' % ci], p['g%d_1_ad' % ci],
                 p['g%d_2_as' % ci], p['g%d_2_ad' % ci],
                 p['g%d_0_b' % ci][None, :], p['g%d_1_b' % ci][None, :],
                 p['g%d_2_b' % ci][None, :]]
        raw_specs += [full((_D, 256)), full((64, 128)),
                      full((32, _H * ncls)),
                      full((_H, 64)), full((_H, 64)),
                      full((_H, 32)), full((_H, 32)),
                      full((_H, ncls)), full((_H, ncls)),
                      full((1, 64)), full((1, 32)), full((1, ncls))]

    out = pl.pallas_call(
        _fused_body,
        grid=(_C, _B),
        in_specs=[
            full((_R, _V)),
            full((_R, _V)),
            full((_V, _D)),
            full((_V, _D)),
            full((1, _D)),
            full((1, _D)),
            pl.BlockSpec((1, _SEQ, _D), lambda i, bi: (bi, 0, 0)),
            full((_D, _H * _DH)),
            full((_D, _H * _DH)),
            full((_D, _H * _DH)),
            full((_H * _DH, _D)),
            byi((_C, _R, 1)),
            full((_NC, _R, _R)),
        ] + raw_specs,
        out_specs=pl.BlockSpec((1, 8, _NPAD), lambda i, bi: (i, 0, 0)),
        out_shape=jax.ShapeDtypeStruct((_C, 8, _NPAD), f32),
        scratch_shapes=[
            pltpu.VMEM((_R, _R), f32),
            pltpu.VMEM((_R, _D), f32),
            pltpu.VMEM((_B, _R, _D), f32),
            pltpu.VMEM((_D, 256), f32),
            pltpu.VMEM((64, 128), f32),
            pltpu.VMEM((32, _H * _NPAD), f32),
            pltpu.VMEM((8, 256), f32),
            pltpu.VMEM((8, 256), f32),
            pltpu.VMEM((8, 128), f32),
            pltpu.VMEM((8, 128), f32),
            pltpu.VMEM((8, 64), f32),
            pltpu.VMEM((8, 64), f32),
            pltpu.VMEM((1, 64), f32),
            pltpu.VMEM((1, 32), f32),
            pltpu.VMEM((1, _NPAD), f32),
        ],
    )(basic_multihot, crucial_multihot, p['Wb'], p['Wc'], bb2, bc2, vis3,
      p['Wq'], p['Wk'], p['Wv'], p['Wo'], maskT, cntp, *raws)

    return jnp.concatenate([out[i][:_B, :_NCLS[i]] for i in range(_C)], axis=1)


# speedup vs baseline: 1.0095x; 1.0095x over previous
import functools

import jax
import jax.numpy as jnp
from jax import lax
from jax.experimental import pallas as pl
from jax.experimental.pallas import tpu as pltpu
from jax.experimental.pallas import tpu_sc as plsc

_V = 5000
_R = 512
_D = 256
_SEQ = 256
_B = 4
_C = 3
_NCLS = (6, 8, 10)
_H = 4
_DH = 64
_E = 131072
_NPAD = 16


def _fused_body(bm, cm, wb, wc, bias, vis, wq, wk, wv, wo,
                mask_r, cnt_r, w0, w1, *rest):
    raw = rest[:30]
    out_ref = rest[30]
    (lncnt_s, rule_s, emb_s, w2p_s, as0_s, ad0_s, as1_s, ad1_s,
     as2_s, ad2_s, b0_s, b1_s, b2_s) = rest[31:]
    f32 = jnp.float32
    i = pl.program_id(0)
    bi = pl.program_id(1)

    @pl.when(jnp.logical_and(i == 0, bi == 0))
    def _():
        w2p_s[...] = jnp.zeros(w2p_s.shape, f32)
        for ref in (as0_s, ad0_s, as1_s, ad1_s, as2_s, ad2_s):
            ref[...] = jnp.zeros(ref.shape, f32)
        for ref in (b0_s, b1_s, b2_s):
            ref[...] = jnp.zeros(ref.shape, f32)

    for ci in range(_C):
        @pl.when(jnp.logical_and(i == ci, bi == 0))
        def _(ci=ci):
            ncls = _NCLS[ci]
            (w2r, as0r, ad0r, as1r, ad1r, as2r, ad2r,
             b0r, b1r, b2r) = raw[ci * 10:(ci + 1) * 10]
            for hh in range(_H):
                w2p_s[:, hh * _NPAD:hh * _NPAD + ncls] = (
                    w2r[:, hh * ncls:(hh + 1) * ncls])
                as0_s[hh:hh + 1, hh * 64:(hh + 1) * 64] = as0r[hh:hh + 1, :]
                ad0_s[hh:hh + 1, hh * 64:(hh + 1) * 64] = ad0r[hh:hh + 1, :]
                as1_s[hh:hh + 1, hh * 32:(hh + 1) * 32] = as1r[hh:hh + 1, :]
                ad1_s[hh:hh + 1, hh * 32:(hh + 1) * 32] = ad1r[hh:hh + 1, :]
                as2_s[hh:hh + 1, hh * _NPAD:hh * _NPAD + ncls] = (
                    as2r[hh:hh + 1, :])
                ad2_s[hh:hh + 1, hh * _NPAD:hh * _NPAD + ncls] = (
                    ad2r[hh:hh + 1, :])
            b0_s[:, :] = b0r[...]
            b1_s[:, :] = b1r[...]
            b2_s[:, :ncls] = b2r[...]

    @pl.when(jnp.logical_and(i == 0, bi == 0))
    def _():
        acc = jnp.dot(bm[...], wb[...], preferred_element_type=f32)
        acc = acc + jnp.dot(cm[...], wc[...], preferred_element_type=f32)
        rule_s[...] = acc + bias[...]

    @pl.when(i == 0)
    def _():
        q = jnp.dot(rule_s[...], wq[...], preferred_element_type=f32)
        visb = vis[0]
        k = jnp.dot(visb, wk[...], preferred_element_type=f32)
        v = jnp.dot(visb, wv[...], preferred_element_type=f32)
        outs = []
        for h in range(_H):
            sl = slice(h * _DH, (h + 1) * _DH)
            logits = lax.dot_general(q[:, sl], k[:, sl],
                                     (((1,), (1,)), ((), ())),
                                     preferred_element_type=f32) * (1.0 / 8.0)
            m = jnp.max(logits, axis=1, keepdims=True)
            p = jnp.exp(logits - m)
            s = jnp.sum(p, axis=1, keepdims=True)
            outs.append(jnp.dot(p / s, v[:, sl], preferred_element_type=f32))
        o = jnp.concatenate(outs, axis=1)
        emb_s[pl.ds(bi, 1)] = jnp.dot(o, wo[...],
                                      preferred_element_type=f32)[None]

    @pl.when(jnp.logical_and(i == 0, bi == 0))
    def _():
        cnt = cnt_r[0]
        for pslab in range(1, cnt_r.shape[0]):
            cnt = cnt + cnt_r[pslab]
        eye = (lax.broadcasted_iota(jnp.int32, (_R, _R), 0) ==
               lax.broadcasted_iota(jnp.int32, (_R, _R), 1)).astype(f32)
        cnt = cnt + eye
        lncnt_s[...] = jnp.where(cnt > 0.0, jnp.log(cnt), -1e30)

    lncnt = lncnt_s[...]

    h = emb_s[pl.ds(bi, 1)][0] * mask_r[0]

    Ws = (w0[0], w1[0], w2p_s[...])
    As = (as0_s[...], as1_s[...], as2_s[...])
    Ad = (ad0_s[...], ad1_s[...], ad2_s[...])
    Bs = (b0_s[...], b1_s[...], b2_s[...])
    dpads = (64, 32, _NPAD)
    for l in range(3):
        W = Ws[l]
        a_s = As[l]
        a_d = Ad[l]
        bvec = Bs[l]
        dpad = dpads[l]
        xp = jnp.dot(h, W, preferred_element_type=f32)
        ed_all = lax.dot_general(xp, a_d, (((1,), (1,)), ((), ())),
                                 preferred_element_type=f32)
        es_all = lax.dot_general(a_s, xp, (((1,), (1,)), ((), ())),
                                 preferred_element_type=f32)
        ones_col = jnp.ones((_R, 1), f32)
        acc = jnp.zeros((_R, dpad), f32)
        for hh in range(_H):
            xpa = jnp.concatenate(
                [xp[:, hh * dpad:(hh + 1) * dpad], ones_col], axis=1)
            t = ed_all[:, hh:hh + 1] + es_all[hh:hh + 1, :]
            e2 = jnp.maximum(t, 0.2 * t) + lncnt
            wgt = jnp.exp(e2)
            od = jnp.dot(wgt, xpa, preferred_element_type=f32)
            acc = acc + od[:, :dpad] / (od[:, dpad:dpad + 1] + 1e-16)
        h = acc * (1.0 / _H) + bvec
        if l < 2:
            h = jnp.maximum(h, 0.0)

    colsum = jnp.sum(h, axis=0, keepdims=True)

    @pl.when(bi == 0)
    def _():
        out_ref[0] = jnp.zeros((8, _NPAD), f32)

    out_ref[0, pl.ds(bi, 1), :] = colsum

    @pl.when(bi == _B - 1)
    def _():
        o = out_ref[0]
        ncls_i = jnp.where(i == 0, _NCLS[0],
                           jnp.where(i == 1, _NCLS[1], _NCLS[2]))
        col = lax.broadcasted_iota(jnp.int32, (1, _NPAD), 1)
        om = jnp.where(col < ncls_i, o, -1e30)
        m = jnp.max(om, axis=1, keepdims=True)
        lse = jnp.log(jnp.sum(jnp.exp(om - m), axis=1, keepdims=True)) + m
        out_ref[0] = o - lse


_NC = 2
_NS = 16
_EPW = _E // (_NC * _NS)
_NJ = _EPW // 128


def _cnt_sc_body(src_hbm, dst_hbm, zeros_hbm, out_hbm,
                 src_v, dst_v, idx_v, vals_v, cnt_sh, sem):
    c = lax.axis_index("c")
    s = lax.axis_index("s")
    wid = c * _NS + s
    base = wid * _EPW
    sl16k = pl.ds(s * (_R * _R // _NS), _R * _R // _NS)

    pltpu.sync_copy(zeros_hbm.at[sl16k], cnt_sh.at[sl16k])

    pltpu.sync_copy(src_hbm.at[pl.ds(base, _EPW)], src_v)
    pltpu.sync_copy(dst_hbm.at[pl.ds(base, _EPW)], dst_v)

    for j in range(_NJ):
        for q in range(8):
            o = j * 128 + q * 16
            d = dst_v[pl.ds(o, 16)]
            sr = src_v[pl.ds(o, 16)]
            idx_v[j, pl.ds(q * 16, 16)] = d * _R + sr
    for q in range(8):
        vals_v[pl.ds(q * 16, 16)] = jnp.full((16,), 1.0, jnp.float32)

    plsc.subcore_barrier()

    descs = [
        pltpu.async_copy(vals_v, cnt_sh.at[idx_v.at[j]], sem, add=True)
        for j in range(_NJ)
    ]
    for d_ in descs:
        d_.wait()

    plsc.subcore_barrier()

    pltpu.sync_copy(cnt_sh.at[sl16k], out_hbm.at[c, sl16k])


def _build_cnt_sc(edge_index):
    src_hbm = edge_index[0]
    dst_hbm = edge_index[1]
    zeros = jnp.zeros((_R * _R,), jnp.float32)
    mesh = plsc.VectorSubcoreMesh(core_axis_name="c", subcore_axis_name="s",
                                  num_cores=_NC, num_subcores=_NS)
    k = functools.partial(
        pl.kernel,
        out_type=jax.ShapeDtypeStruct((_NC, _R * _R), jnp.float32),
        mesh=mesh,
        scratch_types=[
            pltpu.VMEM((_EPW,), jnp.int32),
            pltpu.VMEM((_EPW,), jnp.int32),
            pltpu.VMEM((_NJ, 128), jnp.int32),
            pltpu.VMEM((128,), jnp.float32),
            pltpu.VMEM_SHARED((_R * _R,), jnp.float32),
            pltpu.SemaphoreType.DMA,
        ],
    )(_cnt_sc_body)
    cnt2 = k(src_hbm, dst_hbm, zeros)
    return cnt2.reshape(_NC, _R, _R)


def kernel(vis_emb, params, basic_multihot, crucial_multihot, mask, edge_index):
    p = params
    f32 = jnp.float32
    edge_index = edge_index.astype(jnp.int32)

    bias = (p['bb'] + p['bc'])[None, :]
    vis3 = vis_emb.reshape(_B, _SEQ, _D)
    cntp = _build_cnt_sc(edge_index)

    maskT = mask.reshape(_C, _R, 1)

    w0 = jnp.stack([p['g%d_0_W' % i] for i in range(_C)])
    w1 = jnp.stack([p['g%d_1_W' % i] for i in range(_C)])
    raws = []
    raw_specs = []
    full = lambda shape: pl.BlockSpec(shape, lambda i, bi: (0,) * len(shape))
    byi = lambda shape: pl.BlockSpec((1,) + shape[1:],
                                     lambda i, bi: (i,) + (0,) * (len(shape) - 1))
    for ci in range(_C):
        ncls = _NCLS[ci]
        raws += [p['g%d_2_W' % ci],
                 p['g%d_0_as' % ci], p['g%d_0_ad' % ci],
                 p['g%d_1_as' % ci], p['g%d_1_ad' % ci],
                 p['g%d_2_as' % ci], p['g%d_2_ad' % ci],
                 p['g%d_0_b' % ci][None, :], p['g%d_1_b' % ci][None, :],
                 p['g%d_2_b' % ci][None, :]]
        raw_specs += [full((32, _H * ncls)),
                      full((_H, 64)), full((_H, 64)),
                      full((_H, 32)), full((_H, 32)),
                      full((_H, ncls)), full((_H, ncls)),
                      full((1, 64)), full((1, 32)), full((1, ncls))]

    out = pl.pallas_call(
        _fused_body,
        grid=(_C, _B),
        in_specs=[
            full((_R, _V)),
            full((_R, _V)),
            full((_V, _D)),
            full((_V, _D)),
            full((1, _D)),
            pl.BlockSpec((1, _SEQ, _D), lambda i, bi: (bi, 0, 0)),
            full((_D, _H * _DH)),
            full((_D, _H * _DH)),
            full((_D, _H * _DH)),
            full((_H * _DH, _D)),
            byi((_C, _R, 1)),
            full((_NC, _R, _R)),
            byi((_C, _D, 256)),
            byi((_C, 64, 128)),
        ] + raw_specs,
        out_specs=pl.BlockSpec((1, 8, _NPAD), lambda i, bi: (i, 0, 0)),
        out_shape=jax.ShapeDtypeStruct((_C, 8, _NPAD), f32),
        scratch_shapes=[
            pltpu.VMEM((_R, _R), f32),
            pltpu.VMEM((_R, _D), f32),
            pltpu.VMEM((_B, _R, _D), f32),
            pltpu.VMEM((32, _H * _NPAD), f32),
            pltpu.VMEM((8, 256), f32),
            pltpu.VMEM((8, 256), f32),
            pltpu.VMEM((8, 128), f32),
            pltpu.VMEM((8, 128), f32),
            pltpu.VMEM((8, 64), f32),
            pltpu.VMEM((8, 64), f32),
            pltpu.VMEM((1, 64), f32),
            pltpu.VMEM((1, 32), f32),
            pltpu.VMEM((1, _NPAD), f32),
        ],
    )(basic_multihot, crucial_multihot, p['Wb'], p['Wc'], bias, vis3,
      p['Wq'], p['Wk'], p['Wv'], p['Wo'], maskT, cntp, w0, w1, *raws)

    return jnp.concatenate([out[i][:_B, :_NCLS[i]] for i in range(_C)], axis=1)
